# baseline (speedup 1.0000x reference)
import functools

import jax
import jax.numpy as jnp
from jax import lax
from jax.experimental import pallas as pl
from jax.experimental.pallas import tpu as pltpu
from jax.experimental.pallas import tpu_sc as plsc

N_CLASSES = 100000
EMBED_DIM = 64
BATCH = 16384

NC = 2
NS = 16
LANES = 16
NW = NC * NS
B_PER_W = BATCH // NW
CHUNK = 128
N_CHUNKS = B_PER_W // CHUNK

RB = 256
T_GRID = 196
PACKED_ROWS = T_GRID * RB
SPLIT = PACKED_ROWS


def _transpose_body(lo_ref, hi_ref, o_ref):
    o_ref[:, 0:EMBED_DIM] = lo_ref[...].T
    o_ref[:, EMBED_DIM : 2 * EMBED_DIM] = hi_ref[...].T


def _pack_table(embed_t):
    return pl.pallas_call(
        _transpose_body,
        grid=(T_GRID,),
        in_specs=[
            pl.BlockSpec((EMBED_DIM, RB), lambda j: (0, j)),
            pl.BlockSpec((EMBED_DIM, RB), lambda j: (0, j + T_GRID)),
        ],
        out_specs=pl.BlockSpec((RB, 2 * EMBED_DIM), lambda j: (j, 0)),
        out_shape=jax.ShapeDtypeStruct((PACKED_ROWS, 2 * EMBED_DIM),
                                       jnp.float32),
    )(embed_t, embed_t)


def _gather_kernel(cls_hbm, packed_hbm, out_hbm, idx_v, q_v, rows_v, ot_buf,
                   sem, sem_out):
    wid = lax.axis_index("s") * NC + lax.axis_index("c")
    base = wid * B_PER_W

    pltpu.sync_copy(cls_hbm.at[pl.ds(base, B_PER_W)], idx_v)

    gathers = []
    for g in range(B_PER_W // LANES):
        v = idx_v[pl.ds(g * LANES, LANES)]
        v = jnp.where(v == 0, N_CLASSES - 1, v - 1)
        idx_v[pl.ds(g * LANES, LANES)] = v
        q_v[pl.ds(g * LANES, LANES)] = jnp.where(v >= SPLIT, v - SPLIT, v)
        if g % (CHUNK // LANES) == CHUNK // LANES - 1:
            j = g // (CHUNK // LANES)
            gathers.append(
                pltpu.async_copy(
                    packed_hbm.at[q_v.at[pl.ds(j * CHUNK, CHUNK)]],
                    rows_v.at[pl.ds(j * CHUNK, CHUNK)],
                    sem,
                )
            )
    for c in gathers:
        c.wait()

    lane = lax.iota(jnp.int32, LANES)
    out_copies = []
    for c_hi in range(EMBED_DIM // 8):
        if c_hi >= 2:
            out_copies[c_hi - 2].wait()
        buf = ot_buf.at[c_hi % 2]
        for bc in range(B_PER_W // LANES):
            bvec = bc * LANES + lane
            iv = idx_v[pl.ds(bc * LANES, LANES)]
            col0 = jnp.where(iv >= SPLIT, EMBED_DIM, 0)
            for c in range(c_hi * 8, c_hi * 8 + 8):
                vals = plsc.load_gather(rows_v, [bvec, col0 + c])
                buf[c % 8, pl.ds(bc * LANES, LANES)] = vals
        out_copies.append(
            pltpu.async_copy(
                buf,
                out_hbm.at[pl.ds(c_hi * 8, 8), pl.ds(base, B_PER_W)],
                sem_out,
            )
        )
    out_copies[-2].wait()
    out_copies[-1].wait()


@jax.jit
def kernel(embed, cls):
    packed = _pack_table(embed.T)
    mesh = plsc.VectorSubcoreMesh(core_axis_name="c", subcore_axis_name="s")
    run = functools.partial(
        pl.kernel,
        out_type=jax.ShapeDtypeStruct((EMBED_DIM, BATCH), jnp.float32),
        mesh=mesh,
        scratch_types=[
            pltpu.VMEM((B_PER_W,), jnp.int32),
            pltpu.VMEM((B_PER_W,), jnp.int32),
            pltpu.VMEM((B_PER_W, 2 * EMBED_DIM), jnp.float32),
            pltpu.VMEM((2, 8, B_PER_W), jnp.float32),
            pltpu.SemaphoreType.DMA,
            pltpu.SemaphoreType.DMA,
        ],
        compiler_params=pltpu.CompilerParams(
            use_tc_tiling_on_sc=True, needs_layout_passes=False
        ),
    )(_gather_kernel)
    out_t = run(cls, packed)
    return out_t.T


# speedup vs baseline: 1.7689x; 1.7689x over previous
import functools

import jax
import jax.numpy as jnp
from jax import lax
from jax.experimental import pallas as pl
from jax.experimental.pallas import tpu as pltpu
from jax.experimental.pallas import tpu_sc as plsc

N_CLASSES = 100000
EMBED_DIM = 64
BATCH = 16384

NC = 2
NS = 16
LANES = 16
NW = NC * NS
B_PER_W = BATCH // NW
CHUNK = 128
N_CHUNKS = B_PER_W // CHUNK

RB = 1024
T_GRID = 49
PACKED_ROWS = T_GRID * RB
SPLIT = PACKED_ROWS


def _transpose_body(lo_ref, hi_ref, o_ref):
    o_ref[:, 0:EMBED_DIM] = lo_ref[...].T
    o_ref[:, EMBED_DIM : 2 * EMBED_DIM] = hi_ref[...].T


def _pack_table(embed_t):
    return pl.pallas_call(
        _transpose_body,
        grid=(T_GRID,),
        in_specs=[
            pl.BlockSpec((EMBED_DIM, RB), lambda j: (0, j)),
            pl.BlockSpec((EMBED_DIM, RB), lambda j: (0, j + T_GRID)),
        ],
        out_specs=pl.BlockSpec((RB, 2 * EMBED_DIM), lambda j: (j, 0)),
        out_shape=jax.ShapeDtypeStruct((PACKED_ROWS, 2 * EMBED_DIM),
                                       jnp.float32),
    )(embed_t, embed_t)


def _gather_kernel(cls_hbm, packed_hbm, out_hbm, idx_v, q_v, rows_v, ot_buf,
                   sem, sem_out):
    wid = lax.axis_index("s") * NC + lax.axis_index("c")
    base = wid * B_PER_W

    pltpu.sync_copy(cls_hbm.at[pl.ds(base, B_PER_W)], idx_v)

    gathers = []
    for g in range(B_PER_W // LANES):
        v = idx_v[pl.ds(g * LANES, LANES)]
        v = jnp.where(v == 0, N_CLASSES - 1, v - 1)
        idx_v[pl.ds(g * LANES, LANES)] = v
        q_v[pl.ds(g * LANES, LANES)] = jnp.where(v >= SPLIT, v - SPLIT, v)
        if g % (CHUNK // LANES) == CHUNK // LANES - 1:
            j = g // (CHUNK // LANES)
            gathers.append(
                pltpu.async_copy(
                    packed_hbm.at[q_v.at[pl.ds(j * CHUNK, CHUNK)]],
                    rows_v.at[pl.ds(j * CHUNK, CHUNK)],
                    sem,
                )
            )
    for c in gathers:
        c.wait()

    lane = lax.iota(jnp.int32, LANES)
    out_copies = []
    for c_hi in range(EMBED_DIM // 8):
        if c_hi >= 2:
            out_copies[c_hi - 2].wait()
        buf = ot_buf.at[c_hi % 2]
        for bc in range(B_PER_W // LANES):
            bvec = bc * LANES + lane
            iv = idx_v[pl.ds(bc * LANES, LANES)]
            col0 = jnp.where(iv >= SPLIT, EMBED_DIM, 0)
            for c in range(c_hi * 8, c_hi * 8 + 8):
                vals = plsc.load_gather(rows_v, [bvec, col0 + c])
                buf[c % 8, pl.ds(bc * LANES, LANES)] = vals
        out_copies.append(
            pltpu.async_copy(
                buf,
                out_hbm.at[pl.ds(c_hi * 8, 8), pl.ds(base, B_PER_W)],
                sem_out,
            )
        )
    out_copies[-2].wait()
    out_copies[-1].wait()


@jax.jit
def kernel(embed, cls):
    packed = _pack_table(embed.T)
    mesh = plsc.VectorSubcoreMesh(core_axis_name="c", subcore_axis_name="s")
    run = functools.partial(
        pl.kernel,
        out_type=jax.ShapeDtypeStruct((EMBED_DIM, BATCH), jnp.float32),
        mesh=mesh,
        scratch_types=[
            pltpu.VMEM((B_PER_W,), jnp.int32),
            pltpu.VMEM((B_PER_W,), jnp.int32),
            pltpu.VMEM((B_PER_W, 2 * EMBED_DIM), jnp.float32),
            pltpu.VMEM((2, 8, B_PER_W), jnp.float32),
            pltpu.SemaphoreType.DMA,
            pltpu.SemaphoreType.DMA,
        ],
        compiler_params=pltpu.CompilerParams(
            use_tc_tiling_on_sc=True, needs_layout_passes=False
        ),
    )(_gather_kernel)
    out_t = run(cls, packed)
    return out_t.T
